# baseline (speedup 1.0000x reference)
import functools

import jax
import jax.numpy as jnp
from jax import lax
from jax.experimental import pallas as pl
from jax.experimental.pallas import tpu as pltpu
from jax.experimental.pallas import tpu_sc as plsc

_N = 10000
_E = 320000
_D = 128
_K = 10

_NC = 2
_NS = 16
_NW = _NC * _NS

_C = 128
_KCH = -(-_E // (_NW * _C))
_E_PAD = _NW * _KCH * _C

_HC = 128
_NB = 2
_HKCH = 80
_HE_PAD = _NW * _HKCH * _HC
_IB = 8
_NBLK = _HKCH // _IB
_N_PAD = 10240
_RPT = _N_PAD // _NS
_ZB = 16

_mesh = plsc.VectorSubcoreMesh(core_axis_name="c", subcore_axis_name="s")


@functools.partial(
    pl.kernel,
    mesh=_mesh,
    out_type=jax.ShapeDtypeStruct((_NC, _N_PAD, 16), jnp.float32),
    scratch_types=[
        pltpu.VMEM((_KCH, _C), jnp.int32),
        pltpu.VMEM((_C + _ZB, 16), jnp.float32),
        pltpu.VMEM_SHARED((_N_PAD, 16), jnp.float32),
    ],
)
def _deg(dst_hbm, const_hbm, out_hbm, didx_v, const_v, acc_sh):
    cid = lax.axis_index("c")
    sid = lax.axis_index("s")
    w = cid * _NS + sid
    pltpu.sync_copy(dst_hbm.at[w], didx_v)
    pltpu.sync_copy(const_hbm, const_v)
    base = sid * _RPT
    for i in range(_RPT // _ZB):
        pltpu.sync_copy(const_v.at[pl.ds(_C, _ZB)],
                        acc_sh.at[pl.ds(base + i * _ZB, _ZB)])
    plsc.subcore_barrier()

    def chunk(j, carry):
        pltpu.sync_copy(const_v.at[pl.ds(0, _C)],
                        acc_sh.at[didx_v.at[j]], add=True)
        return carry

    lax.fori_loop(0, _KCH, chunk, 0)
    plsc.subcore_barrier()
    pltpu.sync_copy(acc_sh.at[pl.ds(base, _RPT)],
                    out_hbm.at[cid, pl.ds(base, _RPT)])


@functools.partial(
    pl.kernel,
    mesh=_mesh,
    out_type=jax.ShapeDtypeStruct((_NC, _N_PAD, _D), jnp.float32),
    scratch_types=[
        pltpu.VMEM((2, _IB, _HC), jnp.int32),
        pltpu.VMEM((2, _IB, _HC), jnp.int32),
        pltpu.VMEM((_NB, _HC, _D), jnp.float32),
        pltpu.VMEM_SHARED((_N_PAD, _D), jnp.float32),
        pltpu.SemaphoreType.DMA((2,)),
        pltpu.SemaphoreType.DMA((_NB,)),
        pltpu.SemaphoreType.DMA((_NB,)),
    ],
)
def _hop(m_hbm, src_hbm, dst_hbm, z_hbm, out_hbm,
         sidx_v, didx_v, rows_v, acc_sh, isem, gsem, ssem):
    cid = lax.axis_index("c")
    sid = lax.axis_index("s")
    w = cid * _NS + sid
    base = sid * _RPT

    def load_idx(g, gb):
        pltpu.async_copy(src_hbm.at[w, pl.ds(g * _IB, _IB)],
                         sidx_v.at[gb], isem.at[gb])
        pltpu.async_copy(dst_hbm.at[w, pl.ds(g * _IB, _IB)],
                         didx_v.at[gb], isem.at[gb])

    def wait_idx(gb):
        pltpu.make_async_copy(src_hbm.at[0, pl.ds(0, _IB)],
                              sidx_v.at[gb], isem.at[gb]).wait()
        pltpu.make_async_copy(src_hbm.at[0, pl.ds(0, _IB)],
                              didx_v.at[gb], isem.at[gb]).wait()

    def gather(gb, l, rb):
        pltpu.async_copy(m_hbm.at[sidx_v.at[gb, l]], rows_v.at[rb],
                         gsem.at[rb])

    def wait_gather(rb):
        pltpu.make_async_copy(
            m_hbm.at[pl.ds(0, _HC)], rows_v.at[rb], gsem.at[rb]).wait()

    def scatter(gb, l, rb):
        pltpu.async_copy(rows_v.at[rb], acc_sh.at[didx_v.at[gb, l]],
                         ssem.at[rb], add=True)

    def wait_scatter(rb):
        pltpu.make_async_copy(
            m_hbm.at[pl.ds(0, _HC)], rows_v.at[rb], ssem.at[rb]).wait()

    load_idx(0, 0)
    load_idx(1, 1)
    pltpu.sync_copy(z_hbm, acc_sh.at[pl.ds(base, _RPT)])
    plsc.subcore_barrier()

    def pair(p, carry):
        for gb in range(2):
            g = p * 2 + gb
            wait_idx(gb)
            gather(gb, 0, 0)
            for l in range(_IB):
                rb = l & 1
                if l + 1 < _IB:
                    if l >= 1:
                        wait_scatter(rb ^ 1)
                    gather(gb, l + 1, rb ^ 1)
                wait_gather(rb)
                scatter(gb, l, rb)
            wait_scatter(0)
            wait_scatter(1)

            @pl.when(g + 2 < _NBLK)
            def _():
                load_idx(g + 2, gb)
        return carry

    lax.fori_loop(0, _NBLK // 2, pair, 0)
    plsc.subcore_barrier()
    pltpu.sync_copy(acc_sh.at[pl.ds(base, _RPT)],
                    out_hbm.at[cid, pl.ds(base, _RPT)])


def _norm_body(degp_ref, feats_ref, norm_ref, nsq_ref, m0_ref):
    deg = degp_ref[0, :, 0:1] + degp_ref[1, :, 0:1]
    norm = lax.rsqrt(deg)
    norm_ref[...] = norm
    nsq_ref[...] = norm * norm
    m0_ref[...] = feats_ref[...] * norm


def _combine_body(p_ref, norm_ref, nsq_ref, h_ref, m_ref):
    psum = p_ref[0] + p_ref[1]
    h_ref[...] = psum * norm_ref[...]
    m_ref[...] = psum * nsq_ref[...]


def _final_body(*refs):
    s_ref = refs[0]
    h_refs = refs[1:2 + _K]
    out_ref = refs[2 + _K]
    st = s_ref[...]
    acc = jnp.zeros((_BN, _D), jnp.float32)
    for hr in h_refs:
        hv = hr[...]
        logit = jnp.sum(hv * st, axis=1, keepdims=True)
        sg = 1.0 / (1.0 + jnp.exp(-logit))
        acc = acc + sg * hv
    out_ref[...] = acc


_BN = 1024


def _norm_call(degp, feats_p):
    return pl.pallas_call(
        _norm_body,
        grid=(_N_PAD // _BN,),
        in_specs=[
            pl.BlockSpec((_NC, _BN, 16), lambda i: (0, i, 0)),
            pl.BlockSpec((_BN, _D), lambda i: (i, 0)),
        ],
        out_specs=[
            pl.BlockSpec((_BN, 1), lambda i: (i, 0)),
            pl.BlockSpec((_BN, 1), lambda i: (i, 0)),
            pl.BlockSpec((_BN, _D), lambda i: (i, 0)),
        ],
        out_shape=[
            jax.ShapeDtypeStruct((_N_PAD, 1), jnp.float32),
            jax.ShapeDtypeStruct((_N_PAD, 1), jnp.float32),
            jax.ShapeDtypeStruct((_N_PAD, _D), jnp.float32),
        ],
    )(degp, feats_p)


def _combine_call(p, norm, nsq):
    return pl.pallas_call(
        _combine_body,
        grid=(_N_PAD // _BN,),
        in_specs=[
            pl.BlockSpec((_NC, _BN, _D), lambda i: (0, i, 0)),
            pl.BlockSpec((_BN, 1), lambda i: (i, 0)),
            pl.BlockSpec((_BN, 1), lambda i: (i, 0)),
        ],
        out_specs=[
            pl.BlockSpec((_BN, _D), lambda i: (i, 0)),
            pl.BlockSpec((_BN, _D), lambda i: (i, 0)),
        ],
        out_shape=[
            jax.ShapeDtypeStruct((_N_PAD, _D), jnp.float32),
            jax.ShapeDtypeStruct((_N_PAD, _D), jnp.float32),
        ],
    )(p, norm, nsq)


def _final_call(s_t, hs):
    return pl.pallas_call(
        _final_body,
        grid=(_N_PAD // _BN,),
        in_specs=[pl.BlockSpec((1, _D), lambda i: (0, 0))]
        + [pl.BlockSpec((_BN, _D), lambda i: (i, 0)) for _ in hs],
        out_specs=pl.BlockSpec((_BN, _D), lambda i: (i, 0)),
        out_shape=jax.ShapeDtypeStruct((_N_PAD, _D), jnp.float32),
    )(s_t, *hs)


def kernel(feats, edge_index, s):
    feats = feats.astype(jnp.float32)
    src = edge_index[0].astype(jnp.int32)
    dst = edge_index[1].astype(jnp.int32)

    pad_idx = jnp.full((_E_PAD - _E,), _N, jnp.int32)
    dst_p = jnp.concatenate([dst, pad_idx]).reshape(_NW, _KCH, _C)
    pad_h = jnp.full((_HE_PAD - _E,), _N, jnp.int32)
    src_h = jnp.concatenate([src, pad_h]).reshape(_NW, _HKCH, _HC)
    dst_h = jnp.concatenate([dst, pad_h]).reshape(_NW, _HKCH, _HC)
    z_slab = jnp.zeros((_RPT, _D), jnp.float32)
    feats_p = jnp.pad(feats, ((0, _N_PAD - _N), (0, 0)))

    const16 = jnp.concatenate(
        [jnp.ones((_C, 16), jnp.float32), jnp.zeros((_ZB, 16), jnp.float32)])
    z128 = jnp.zeros((_ZB, _D), jnp.float32)

    degp = _deg(dst_p, const16)
    norm, nsq, m = _norm_call(degp, feats_p)

    hs = [feats_p]
    for _ in range(_K):
        p = _hop(m, src_h, dst_h, z_slab)
        h, m = _combine_call(p, norm, nsq)
        hs.append(h)

    out = _final_call(jnp.transpose(s), hs)
    return out[:_N]
